# baseline (speedup 1.0000x reference)
import jax
import jax.numpy as jnp
from jax.experimental import pallas as pl
from jax.experimental.pallas import tpu as pltpu

_B, _S, _DIN, _DOUT, _E = 2, 4096, 2048, 2048, 8
_R = 64
_RP = _R // _E
_SCALING = 16.0 / _R

_ST = 512
_WCH = 256
_NCH = _DOUT // _WCH


def _moe_lora_kernel(emb_ref, rw_ref, a_ref, b2_ref, w_hbm, x_ref,
                     o_ref, weff_ref, land_ref, *sems):
    @pl.when(pl.program_id(0) == 0)
    def _():
        copies = []
        for k in range(_NCH):
            rows = pl.ds(k * _WCH, _WCH)
            cp = pltpu.make_async_copy(
                w_hbm.at[rows, :], land_ref.at[rows, :], sems[k])
            cp.start()
            copies.append(cp)
        logits = jnp.sum(rw_ref[...] * emb_ref[...], axis=1, keepdims=True)
        m = jnp.max(logits)
        p = jnp.exp(logits - m)
        probs = p / jnp.sum(p)
        i_idx = jax.lax.broadcasted_iota(jnp.int32, (_R, _E), 0) // _RP
        e_idx = jax.lax.broadcasted_iota(jnp.int32, (_R, _E), 1)
        onehot = (i_idx == e_idx).astype(jnp.float32)
        s64 = jax.lax.dot_general(
            onehot, probs, (((1,), (0,)), ((), ())),
            preferred_element_type=jnp.float32)
        a_scaled = a_ref[...] * (s64 * _SCALING)
        for k in range(_NCH):
            rows = pl.ds(k * _WCH, _WCH)
            delta = jax.lax.dot_general(
                b2_ref[rows, :], a_scaled, (((1,), (0,)), ((), ())),
                preferred_element_type=jnp.float32)
            copies[k].wait()
            weff_ref[:, rows] = jnp.transpose(
                land_ref[rows, :] + delta, (1, 0)).astype(jnp.bfloat16)
            o_ref[:, rows] = jax.lax.dot_general(
                x_ref[...].astype(jnp.bfloat16), weff_ref[:, rows],
                (((1,), (0,)), ((), ())),
                preferred_element_type=jnp.float32)

    @pl.when(pl.program_id(0) > 0)
    def _():
        o_ref[...] = jax.lax.dot_general(
            x_ref[...].astype(jnp.bfloat16), weff_ref[...],
            (((1,), (0,)), ((), ())),
            preferred_element_type=jnp.float32)


def kernel(x, task_ids, base_W, loraA, loraB, expert_emb, router_W):
    del task_ids
    xf = x.reshape(_B * _S, _DIN)
    a_all = loraA.reshape(_R, _DIN)
    b2 = loraB.transpose(1, 0, 2).reshape(_DOUT, _R)
    grid = ((_B * _S) // _ST,)
    out = pl.pallas_call(
        _moe_lora_kernel,
        grid=grid,
        in_specs=[
            pl.BlockSpec((1, _DOUT), lambda s: (0, 0)),
            pl.BlockSpec((_E, _DOUT), lambda s: (0, 0)),
            pl.BlockSpec((_R, _DIN), lambda s: (0, 0)),
            pl.BlockSpec((_DOUT, _R), lambda s: (0, 0)),
            pl.BlockSpec(memory_space=pl.ANY),
            pl.BlockSpec((_ST, _DIN), lambda s: (s, 0)),
        ],
        out_specs=pl.BlockSpec((_ST, _DOUT), lambda s: (s, 0)),
        out_shape=jax.ShapeDtypeStruct((_B * _S, _DOUT), jnp.float32),
        scratch_shapes=[pltpu.VMEM((_DIN, _DOUT), jnp.bfloat16),
                        pltpu.VMEM((_DOUT, _DIN), jnp.float32)]
        + [pltpu.SemaphoreType.DMA] * _NCH,
        compiler_params=pltpu.CompilerParams(
            dimension_semantics=("arbitrary",)),
    )(expert_emb, router_W, a_all, b2, base_W, xf)
    return out.reshape(_B, _S, _DOUT)


# speedup vs baseline: 1.0298x; 1.0298x over previous
import jax
import jax.numpy as jnp
from jax.experimental import pallas as pl
from jax.experimental.pallas import tpu as pltpu

_B, _S, _DIN, _DOUT, _E = 2, 4096, 2048, 2048, 8
_R = 64
_RP = _R // _E
_SCALING = 16.0 / _R

_ST = 512
_WCH = 512
_NCH = _DOUT // _WCH


def _moe_lora_kernel(emb_ref, rw_ref, a_ref, b2_ref, w_hbm, x_ref,
                     o_ref, weff_ref, land_ref, *sems):
    @pl.when(pl.program_id(0) == 0)
    def _():
        copies = []
        for k in range(_NCH):
            rows = pl.ds(k * _WCH, _WCH)
            cp = pltpu.make_async_copy(
                w_hbm.at[rows, :], land_ref.at[rows, :], sems[k])
            cp.start()
            copies.append(cp)
        logits = jnp.sum(rw_ref[...] * emb_ref[...], axis=1, keepdims=True)
        m = jnp.max(logits)
        p = jnp.exp(logits - m)
        probs = p / jnp.sum(p)
        i_idx = jax.lax.broadcasted_iota(jnp.int32, (_R, _E), 0) // _RP
        e_idx = jax.lax.broadcasted_iota(jnp.int32, (_R, _E), 1)
        onehot = (i_idx == e_idx).astype(jnp.float32)
        s64 = jax.lax.dot_general(
            onehot, probs, (((1,), (0,)), ((), ())),
            preferred_element_type=jnp.float32)
        a_scaled = a_ref[...] * (s64 * _SCALING)
        for k in range(_NCH):
            rows = pl.ds(k * _WCH, _WCH)
            delta = jax.lax.dot_general(
                b2_ref[rows, :], a_scaled, (((1,), (0,)), ((), ())),
                preferred_element_type=jnp.float32)
            copies[k].wait()
            weff_ref[:, rows] = jnp.transpose(
                land_ref[rows, :] + delta, (1, 0)).astype(jnp.bfloat16)
            o_ref[:, rows] = jax.lax.dot_general(
                x_ref[...].astype(jnp.bfloat16), weff_ref[:, rows],
                (((1,), (0,)), ((), ())),
                preferred_element_type=jnp.float32)

    @pl.when(pl.program_id(0) > 0)
    def _():
        o_ref[...] = jax.lax.dot_general(
            x_ref[...].astype(jnp.bfloat16), weff_ref[...],
            (((1,), (0,)), ((), ())),
            preferred_element_type=jnp.float32)


def kernel(x, task_ids, base_W, loraA, loraB, expert_emb, router_W):
    del task_ids
    xf = x.reshape(_B * _S, _DIN)
    a_all = loraA.reshape(_R, _DIN)
    b2 = loraB.transpose(1, 0, 2).reshape(_DOUT, _R)
    grid = ((_B * _S) // _ST,)
    out = pl.pallas_call(
        _moe_lora_kernel,
        grid=grid,
        in_specs=[
            pl.BlockSpec((1, _DOUT), lambda s: (0, 0)),
            pl.BlockSpec((_E, _DOUT), lambda s: (0, 0)),
            pl.BlockSpec((_R, _DIN), lambda s: (0, 0)),
            pl.BlockSpec((_DOUT, _R), lambda s: (0, 0)),
            pl.BlockSpec(memory_space=pl.ANY),
            pl.BlockSpec((_ST, _DIN), lambda s: (s, 0)),
        ],
        out_specs=pl.BlockSpec((_ST, _DOUT), lambda s: (s, 0)),
        out_shape=jax.ShapeDtypeStruct((_B * _S, _DOUT), jnp.float32),
        scratch_shapes=[pltpu.VMEM((_DIN, _DOUT), jnp.bfloat16),
                        pltpu.VMEM((_DOUT, _DIN), jnp.float32)]
        + [pltpu.SemaphoreType.DMA] * _NCH,
        compiler_params=pltpu.CompilerParams(
            dimension_semantics=("arbitrary",)),
    )(expert_emb, router_W, a_all, b2, base_W, xf)
    return out.reshape(_B, _S, _DOUT)
